# baseline (speedup 1.0000x reference)
import functools

import jax
import jax.numpy as jnp
from jax import lax
from jax.experimental import pallas as pl
from jax.experimental.pallas import tpu as pltpu
from jax.experimental.pallas import tpu_sc as plsc

N = 10000
E = 320000
D_IN = 128
H = 64

NC = 2
NS = 16
NW = NC * NS

CH = 80
NCHUNK_TOTAL = E // CH
CPW = NCHUNK_TOTAL // NW
STRIPE = N // NS
NSTRIPE = NS

_mesh = plsc.VectorSubcoreMesh(
    core_axis_name="c", subcore_axis_name="s", num_cores=NC, num_subcores=NS)
_sc_params = pltpu.CompilerParams(use_tc_tiling_on_sc=False)


def _worker_id():
  c = lax.axis_index("c")
  s = lax.axis_index("s")
  return c * NS + s, c, s


@functools.partial(
    pl.kernel,
    out_type=jax.ShapeDtypeStruct((NC, N, 16), jnp.float32),
    mesh=_mesh,
    scratch_types=[
        pltpu.VMEM((CPW, CH), jnp.int32),
        pltpu.VMEM((CH, 16), jnp.float32),
        pltpu.VMEM_SHARED((N, 16), jnp.float32),
        pltpu.SemaphoreType.DMA,
        pltpu.SemaphoreType.DMA,
        pltpu.SemaphoreType.DMA,
    ],
    compiler_params=_sc_params,
)
def _sc_degree(dst_hbm, zeros_hbm, out_hbm, dst_v, ones_v, acc_sh,
               sem, sem2, sem3):
  wid, c, s = _worker_id()

  ld = pltpu.make_async_copy(dst_hbm.at[wid], dst_v, sem2)
  ld.start()
  zfill = pltpu.make_async_copy(
      zeros_hbm.at[pl.ds(s * STRIPE, STRIPE)],
      acc_sh.at[pl.ds(s * STRIPE, STRIPE)], sem3)
  zfill.start()

  def fill(i, _):
    ones_v[i, :] = jnp.full((16,), 1.0, dtype=jnp.float32)
    return ()
  lax.fori_loop(0, CH, fill, ())

  ld.wait()
  zfill.wait()
  plsc.subcore_barrier()

  GROUP = 25
  def group(gbase, _):
    descs = []
    for k in range(GROUP):
      d = pltpu.make_async_copy(
          ones_v, acc_sh.at[dst_v.at[gbase + k]], sem)
      d.start(add=True)
      descs.append(d)
    for d in descs:
      d.wait()
    return ()
  lax.fori_loop(0, CPW // GROUP, lambda i, cr: group(i * GROUP, cr), ())

  plsc.subcore_barrier()
  @pl.when(s < NSTRIPE)
  def _():
    pltpu.sync_copy(acc_sh.at[pl.ds(s * STRIPE, STRIPE)],
                    out_hbm.at[c, pl.ds(s * STRIPE, STRIPE)])


@functools.partial(
    pl.kernel,
    out_type=jax.ShapeDtypeStruct((NC, N, H), jnp.float32),
    mesh=_mesh,
    scratch_types=[
        pltpu.VMEM((CPW, CH), jnp.int32),
        pltpu.VMEM((CPW, CH), jnp.int32),
    ] + [pltpu.VMEM((CH, H), jnp.float32)] * 5
    + [pltpu.VMEM_SHARED((N, H), jnp.float32)]
    + [pltpu.SemaphoreType.DMA] * 10,
    compiler_params=_sc_params,
)
def _sc_segment_sum(src_hbm, dst_hbm, g_hbm, zeros_hbm, out_hbm,
                    src_v, dst_v, r0, r1, r2, r3, r4, acc_sh,
                    g0, g1s, g2s, g3, g4, s0, s1, s2, s3, s4):
  wid, c, s = _worker_id()
  rows = [r0, r1, r2, r3, r4]
  gsem = [g0, g1s, g2s, g3, g4]
  ssem = [s0, s1, s2, s3, s4]

  def gather_start(j, k):
    pltpu.make_async_copy(g_hbm.at[src_v.at[j]], rows[k], gsem[k]).start()

  def gather_wait(k):
    pltpu.make_async_copy(g_hbm.at[src_v.at[0]], rows[k], gsem[k]).wait()

  def scatter_start(j, k):
    pltpu.make_async_copy(
        rows[k], acc_sh.at[dst_v.at[j]], ssem[k]).start(add=True)

  def scatter_wait(k):
    pltpu.make_async_copy(
        rows[k], acc_sh.at[dst_v.at[0]], ssem[k]).wait()

  ld_src = pltpu.make_async_copy(src_hbm.at[wid], src_v, ssem[0])
  ld_dst = pltpu.make_async_copy(dst_hbm.at[wid], dst_v, ssem[1])
  ld_src.start()
  ld_dst.start()
  zfill = pltpu.make_async_copy(
      zeros_hbm.at[pl.ds(s * STRIPE, STRIPE)],
      acc_sh.at[pl.ds(s * STRIPE, STRIPE)], ssem[2])
  zfill.start()
  ld_src.wait()
  ld_dst.wait()

  gather_start(0, 0)
  gather_start(1, 1)
  gather_start(2, 2)
  zfill.wait()
  plsc.subcore_barrier()

  def block(jj, _):
    for i in range(5):
      j = jj * 5 + i
      kf = (i + 3) % 5
      @pl.when(j >= 2)
      def _():
        scatter_wait(kf)
      @pl.when(j + 3 < CPW)
      def _():
        gather_start(j + 3, kf)
      gather_wait(i)
      scatter_start(j, i)
    return ()
  lax.fori_loop(0, CPW // 5, block, ())
  scatter_wait((CPW - 2) % 5)
  scatter_wait((CPW - 1) % 5)

  plsc.subcore_barrier()
  @pl.when(s < NSTRIPE)
  def _():
    pltpu.sync_copy(acc_sh.at[pl.ds(s * STRIPE, STRIPE)],
                    out_hbm.at[c, pl.ds(s * STRIPE, STRIPE)])


NP2 = N // 2


def _pair(h):
  h3 = jnp.reshape(h, (NP2, 2, H))
  return jnp.concatenate([h3[:, 0, :], h3[:, 1, :]], axis=1)


def _unpair(hp):
  a = hp[:, :H]
  b = hp[:, H:]
  return jnp.reshape(
      jnp.concatenate([a[:, None, :], b[:, None, :]], axis=1), (N, H))


def _tc_matmul1(x_ref, w1_ref, u_ref):
  x3 = jnp.reshape(x_ref[...], (NP2, 2, D_IN))
  ge = jnp.dot(x3[:, 0, :], w1_ref[...], preferred_element_type=jnp.float32)
  go = jnp.dot(x3[:, 1, :], w1_ref[...], preferred_element_type=jnp.float32)
  u_ref[...] = jnp.concatenate([ge, go], axis=1)


def _tc_first(deg_ref, u_ref, g1_ref, dinv_ref):
  d = deg_ref[0] + deg_ref[1]
  variants = []
  for q in range(4):
    a = d[:, 32 * q: 32 * q + 16]
    b = d[:, 32 * q + 16: 32 * q + 32]
    row = jnp.concatenate([a, a, a, a, b, b, b, b], axis=1)
    variants.append(row[:, None, :])
  degp = jnp.reshape(jnp.concatenate(variants, axis=1), (NP2, 2 * H))
  dinvp = lax.rsqrt(degp + 1.0)
  dinv_ref[...] = dinvp
  g1_ref[...] = u_ref[...] * dinvp


def _tc_mid(acc_ref, g1_ref, dinv_ref, b1_ref, w2blk_ref, g2_ref):
  dinv = dinv_ref[...]
  tot = acc_ref[0] + acc_ref[1] + g1_ref[...]
  h1 = jnp.maximum(dinv * tot + b1_ref[...], 0.0)
  h = jnp.dot(h1, w2blk_ref[...], preferred_element_type=jnp.float32)
  g2_ref[...] = h * dinv


def _sigmoid(v):
  return 1.0 / (1.0 + jnp.exp(-v))


def _tc_last(acc_ref, g2_ref, dinv_ref, b2_ref, wgates_ref,
             bih_ref, bhh_ref, wp_ref, bp_ref, out_ref, hnew_ref):
  dinv = dinv_ref[...]
  tot = acc_ref[0] + acc_ref[1] + g2_ref[...]
  h2 = jnp.maximum(dinv * tot + b2_ref[...], 0.0)
  gi = jnp.dot(h2, wgates_ref[...], preferred_element_type=jnp.float32)
  bih = bih_ref[...]
  bhh = bhh_ref[...]
  r = _sigmoid(gi[:, 0:128] + bih[0:1, :] + bhh[0:1, :])
  z = _sigmoid(gi[:, 128:256] + bih[1:2, :] + bhh[1:2, :])
  n = jnp.tanh(gi[:, 256:384] + bih[2:3, :] + r * bhh[2:3, :])
  hnew = _unpair((1.0 - z) * n)
  hnew_ref[0] = hnew
  out_ref[...] = jnp.dot(hnew, wp_ref[...],
                         preferred_element_type=jnp.float32) + bp_ref[...]


def kernel(x, edge_index, W1, b1, W2, b2, Wih, Whh, bih, bhh, Wp, bp):
  del Whh

  dst = edge_index[1].reshape(NW, CPW, CH)
  src_rows = lax.optimization_barrier(edge_index)[0]
  src = src_rows.reshape(NW, CPW, CH)
  zeros16 = jnp.zeros((N, 16), jnp.float32)
  zerosH = jnp.zeros((N, H), jnp.float32)

  deg_p = _sc_degree(dst, zeros16)

  def blkdiag(w):
    z = jnp.zeros_like(w)
    return jnp.concatenate(
        [jnp.concatenate([w, z], axis=1), jnp.concatenate([z, w], axis=1)],
        axis=0)

  def pair_cols(v, rows):
    v = v.reshape(rows, H)
    return jnp.concatenate([v, v], axis=1)

  vmem = pltpu.VMEM
  u1p = pl.pallas_call(
      _tc_matmul1,
      out_shape=jax.ShapeDtypeStruct((NP2, 2 * H), jnp.float32),
      in_specs=[pl.BlockSpec(memory_space=vmem)] * 2,
      out_specs=pl.BlockSpec(memory_space=vmem),
  )(x, W1)

  g1p, dinvp = pl.pallas_call(
      _tc_first,
      out_shape=(jax.ShapeDtypeStruct((NP2, 2 * H), jnp.float32),
                 jax.ShapeDtypeStruct((NP2, 2 * H), jnp.float32)),
      in_specs=[pl.BlockSpec(memory_space=vmem)] * 2,
      out_specs=(pl.BlockSpec(memory_space=vmem),
                 pl.BlockSpec(memory_space=vmem)),
  )(deg_p.reshape(NC, N * 16 // 128, 128), u1p)

  acc1 = _sc_segment_sum(src, dst, g1p.reshape(N, H), zerosH)

  g2p = pl.pallas_call(
      _tc_mid,
      out_shape=jax.ShapeDtypeStruct((NP2, 2 * H), jnp.float32),
      in_specs=[pl.BlockSpec(memory_space=vmem)] * 5,
      out_specs=pl.BlockSpec(memory_space=vmem),
  )(acc1.reshape(NC, NP2, 2 * H), g1p, dinvp, pair_cols(b1, 1), blkdiag(W2))

  acc2 = _sc_segment_sum(src, dst, g2p.reshape(N, H), zerosH)

  out, hnew = pl.pallas_call(
      _tc_last,
      out_shape=(jax.ShapeDtypeStruct((N, 2), jnp.float32),
                 jax.ShapeDtypeStruct((1, N, H), jnp.float32)),
      in_specs=[pl.BlockSpec(memory_space=vmem)] * 9,
      out_specs=(pl.BlockSpec(memory_space=vmem),
                 pl.BlockSpec(memory_space=vmem)),
  )(acc2.reshape(NC, NP2, 2 * H), g2p, dinvp, pair_cols(b2, 1),
    jnp.concatenate([blkdiag(Wih[0:H].T), blkdiag(Wih[H:2 * H].T),
                     blkdiag(Wih[2 * H:].T)], axis=1),
    pair_cols(bih, 3), pair_cols(bhh, 3), Wp.T, bp.reshape(1, 2))

  return (out, hnew)


# speedup vs baseline: 1.0618x; 1.0618x over previous
import functools

import jax
import jax.numpy as jnp
from jax import lax
from jax.experimental import pallas as pl
from jax.experimental.pallas import tpu as pltpu
from jax.experimental.pallas import tpu_sc as plsc

N = 10000
E = 320000
D_IN = 128
H = 64

NC = 2
NS = 16
NW = NC * NS

CH = 80
NCHUNK_TOTAL = E // CH
CPW = NCHUNK_TOTAL // NW
STRIPE = N // NS

_mesh = plsc.VectorSubcoreMesh(
    core_axis_name="c", subcore_axis_name="s", num_cores=NC, num_subcores=NS)
_sc_params = pltpu.CompilerParams(use_tc_tiling_on_sc=False)


def _worker_id():
  c = lax.axis_index("c")
  s = lax.axis_index("s")
  return c * NS + s, c, s


@functools.partial(
    pl.kernel,
    out_type=jax.ShapeDtypeStruct((NC, N, 16), jnp.float32),
    mesh=_mesh,
    scratch_types=[
        pltpu.VMEM((CPW, CH), jnp.int32),
        pltpu.VMEM((CH, 16), jnp.float32),
        pltpu.VMEM_SHARED((N, 16), jnp.float32),
        pltpu.SemaphoreType.DMA,
        pltpu.SemaphoreType.DMA,
        pltpu.SemaphoreType.DMA,
    ],
    compiler_params=_sc_params,
)
def _sc_degree(dst_hbm, zeros_hbm, out_hbm, dst_v, ones_v, acc_sh,
               sem, sem2, sem3):
  wid, c, s = _worker_id()

  ld = pltpu.make_async_copy(dst_hbm.at[wid], dst_v, sem2)
  ld.start()
  zfill = pltpu.make_async_copy(
      zeros_hbm.at[pl.ds(s * STRIPE, STRIPE)],
      acc_sh.at[pl.ds(s * STRIPE, STRIPE)], sem3)
  zfill.start()

  def fill(i, _):
    ones_v[i, :] = jnp.full((16,), 1.0, dtype=jnp.float32)
    return ()
  lax.fori_loop(0, CH, fill, ())

  ld.wait()
  zfill.wait()
  plsc.subcore_barrier()

  GROUP = 25
  def group(gbase, _):
    descs = []
    for k in range(GROUP):
      d = pltpu.make_async_copy(
          ones_v, acc_sh.at[dst_v.at[gbase + k]], sem)
      d.start(add=True)
      descs.append(d)
    for d in descs:
      d.wait()
    return ()
  lax.fori_loop(0, CPW // GROUP, lambda i, cr: group(i * GROUP, cr), ())

  plsc.subcore_barrier()
  pltpu.sync_copy(acc_sh.at[pl.ds(s * STRIPE, STRIPE)],
                  out_hbm.at[c, pl.ds(s * STRIPE, STRIPE)])


@functools.partial(
    pl.kernel,
    out_type=jax.ShapeDtypeStruct((NC, N, H), jnp.float32),
    mesh=_mesh,
    scratch_types=[
        pltpu.VMEM((CPW, CH), jnp.int32),
        pltpu.VMEM((CPW, CH), jnp.int32),
    ] + [pltpu.VMEM((CH, H), jnp.float32)] * 5
    + [pltpu.VMEM_SHARED((N, H), jnp.float32)]
    + [pltpu.SemaphoreType.DMA] * 10,
    compiler_params=_sc_params,
)
def _sc_segment_sum(src_hbm, dst_hbm, g_hbm, zeros_hbm, out_hbm,
                    src_v, dst_v, r0, r1, r2, r3, r4, acc_sh,
                    g0, g1s, g2s, g3, g4, s0, s1, s2, s3, s4):
  wid, c, s = _worker_id()
  rows = [r0, r1, r2, r3, r4]
  gsem = [g0, g1s, g2s, g3, g4]
  ssem = [s0, s1, s2, s3, s4]

  def gather_start(j, k):
    pltpu.make_async_copy(g_hbm.at[src_v.at[j]], rows[k], gsem[k]).start()

  def gather_wait(k):
    pltpu.make_async_copy(g_hbm.at[src_v.at[0]], rows[k], gsem[k]).wait()

  def scatter_start(j, k):
    pltpu.make_async_copy(
        rows[k], acc_sh.at[dst_v.at[j]], ssem[k]).start(add=True)

  def scatter_wait(k):
    pltpu.make_async_copy(
        rows[k], acc_sh.at[dst_v.at[0]], ssem[k]).wait()

  ld_src = pltpu.make_async_copy(src_hbm.at[wid], src_v, ssem[0])
  ld_dst = pltpu.make_async_copy(dst_hbm.at[wid], dst_v, ssem[1])
  ld_src.start()
  ld_dst.start()
  zfill = pltpu.make_async_copy(
      zeros_hbm.at[pl.ds(s * STRIPE, STRIPE)],
      acc_sh.at[pl.ds(s * STRIPE, STRIPE)], ssem[2])
  zfill.start()
  ld_src.wait()
  ld_dst.wait()

  gather_start(0, 0)
  gather_start(1, 1)
  gather_start(2, 2)
  zfill.wait()
  plsc.subcore_barrier()

  def block(jj, _):
    for i in range(5):
      j = jj * 5 + i
      kf = (i + 3) % 5
      @pl.when(j >= 2)
      def _():
        scatter_wait(kf)
      @pl.when(j + 3 < CPW)
      def _():
        gather_start(j + 3, kf)
      gather_wait(i)
      scatter_start(j, i)
    return ()
  lax.fori_loop(0, CPW // 5, block, ())
  scatter_wait((CPW - 2) % 5)
  scatter_wait((CPW - 1) % 5)

  plsc.subcore_barrier()
  pltpu.sync_copy(acc_sh.at[pl.ds(s * STRIPE, STRIPE)],
                  out_hbm.at[c, pl.ds(s * STRIPE, STRIPE)])


NP2 = N // 2


def _unpair(hp):
  a = hp[:, :H]
  b = hp[:, H:]
  return jnp.reshape(
      jnp.concatenate([a[:, None, :], b[:, None, :]], axis=1), (N, H))


def _tc_matmul1(x_ref, w1_ref, u_ref):
  x3 = jnp.reshape(x_ref[...], (NP2, 2, D_IN))
  ge = jnp.dot(x3[:, 0, :], w1_ref[...], preferred_element_type=jnp.float32)
  go = jnp.dot(x3[:, 1, :], w1_ref[...], preferred_element_type=jnp.float32)
  u_ref[...] = jnp.concatenate([ge, go], axis=1)


def _tc_first(deg_ref, u_ref, g1_ref, dinv_ref):
  d = deg_ref[0] + deg_ref[1]
  variants = []
  for q in range(4):
    a = d[:, 32 * q: 32 * q + 16]
    b = d[:, 32 * q + 16: 32 * q + 32]
    row = jnp.concatenate([a, a, a, a, b, b, b, b], axis=1)
    variants.append(row[:, None, :])
  degp = jnp.reshape(jnp.concatenate(variants, axis=1), (NP2, 2 * H))
  dinvp = lax.rsqrt(degp + 1.0)
  dinv_ref[...] = dinvp
  g1_ref[...] = u_ref[...] * dinvp


def _tc_mid(acc_ref, g1_ref, dinv_ref, b1_ref, w2blk_ref, g2_ref):
  dinv = dinv_ref[...]
  tot = acc_ref[0] + acc_ref[1] + g1_ref[...]
  h1 = jnp.maximum(dinv * tot + b1_ref[...], 0.0)
  h = jnp.dot(h1, w2blk_ref[...], preferred_element_type=jnp.float32)
  g2_ref[...] = h * dinv


def _sigmoid(v):
  return 1.0 / (1.0 + jnp.exp(-v))


def _tc_last(acc_ref, g2_ref, dinv_ref, b2_ref, wgates_ref,
             bih_ref, bhh_ref, wp_ref, bp_ref, out_ref, hnew_ref):
  dinv = dinv_ref[...]
  tot = acc_ref[0] + acc_ref[1] + g2_ref[...]
  h2 = jnp.maximum(dinv * tot + b2_ref[...], 0.0)
  gi = jnp.dot(h2, wgates_ref[...], preferred_element_type=jnp.float32)
  bih = bih_ref[...]
  bhh = bhh_ref[...]
  r = _sigmoid(gi[:, 0:128] + bih[0:1, :] + bhh[0:1, :])
  z = _sigmoid(gi[:, 128:256] + bih[1:2, :] + bhh[1:2, :])
  n = jnp.tanh(gi[:, 256:384] + bih[2:3, :] + r * bhh[2:3, :])
  hnew = _unpair((1.0 - z) * n)
  hnew_ref[0] = hnew
  out_ref[...] = jnp.dot(hnew, wp_ref[...],
                         preferred_element_type=jnp.float32) + bp_ref[...]


def kernel(x, edge_index, W1, b1, W2, b2, Wih, Whh, bih, bhh, Wp, bp):
  del Whh

  src = edge_index[0].reshape(NW, CPW, CH)
  dst = edge_index[1].reshape(NW, CPW, CH)
  zeros16 = jnp.zeros((N, 16), jnp.float32)
  zerosH = jnp.zeros((N, H), jnp.float32)

  deg_p = _sc_degree(dst, zeros16)

  def blkdiag(w):
    z = jnp.zeros_like(w)
    return jnp.concatenate(
        [jnp.concatenate([w, z], axis=1), jnp.concatenate([z, w], axis=1)],
        axis=0)

  def pair_cols(v, rows):
    v = v.reshape(rows, H)
    return jnp.concatenate([v, v], axis=1)

  vmem = pltpu.VMEM
  u1p = pl.pallas_call(
      _tc_matmul1,
      out_shape=jax.ShapeDtypeStruct((NP2, 2 * H), jnp.float32),
      in_specs=[pl.BlockSpec(memory_space=vmem)] * 2,
      out_specs=pl.BlockSpec(memory_space=vmem),
  )(x, W1)

  g1p, dinvp = pl.pallas_call(
      _tc_first,
      out_shape=(jax.ShapeDtypeStruct((NP2, 2 * H), jnp.float32),
                 jax.ShapeDtypeStruct((NP2, 2 * H), jnp.float32)),
      in_specs=[pl.BlockSpec(memory_space=vmem)] * 2,
      out_specs=(pl.BlockSpec(memory_space=vmem),
                 pl.BlockSpec(memory_space=vmem)),
  )(deg_p.reshape(NC, N * 16 // 128, 128), u1p)

  acc1 = _sc_segment_sum(src, dst, g1p.reshape(N, H), zerosH)

  g2p = pl.pallas_call(
      _tc_mid,
      out_shape=jax.ShapeDtypeStruct((NP2, 2 * H), jnp.float32),
      in_specs=[pl.BlockSpec(memory_space=vmem)] * 5,
      out_specs=pl.BlockSpec(memory_space=vmem),
  )(acc1.reshape(NC, NP2, 2 * H), g1p, dinvp, pair_cols(b1, 1), blkdiag(W2))

  acc2 = _sc_segment_sum(src, dst, g2p.reshape(N, H), zerosH)

  out, hnew = pl.pallas_call(
      _tc_last,
      out_shape=(jax.ShapeDtypeStruct((N, 2), jnp.float32),
                 jax.ShapeDtypeStruct((1, N, H), jnp.float32)),
      in_specs=[pl.BlockSpec(memory_space=vmem)] * 9,
      out_specs=(pl.BlockSpec(memory_space=vmem),
                 pl.BlockSpec(memory_space=vmem)),
  )(acc2.reshape(NC, NP2, 2 * H), g2p, dinvp, pair_cols(b2, 1),
    jnp.concatenate([blkdiag(Wih[0:H].T), blkdiag(Wih[H:2 * H].T),
                     blkdiag(Wih[2 * H:].T)], axis=1),
    pair_cols(bih, 3), pair_cols(bhh, 3), Wp.T, bp.reshape(1, 2))

  return (out, hnew)
